# baseline (speedup 1.0000x reference)
import functools

import jax
import jax.numpy as jnp
from jax import lax
from jax.experimental import pallas as pl
from jax.experimental.pallas import tpu as pltpu
from jax.experimental.pallas import tpu_sc as plsc

BATCH = 16384
ACTIONS = 16
STATES = 1000000
_PACK = 128 // ACTIONS

_info = plsc.get_sparse_core_info()
_NC, _NS, _L = _info.num_cores, _info.num_subcores, _info.num_lanes
_NW = _NC * _NS
_B_PER_W = BATCH // _NW
_CHUNK = 128
_NCHUNK = _B_PER_W // _CHUNK
_OUT_PER_W = _B_PER_W // _PACK


def _sc_body(x_hbm, table_hbm, out_hbm, idx_flat, rowsel, big_v, out_s, sem):
    wid = lax.axis_index("s") * _NC + lax.axis_index("c")
    base = wid * _B_PER_W

    pltpu.sync_copy(x_hbm.at[pl.ds(base, _B_PER_W)], idx_flat)

    for i in range(_B_PER_W // _L):
        v = idx_flat[pl.ds(i * _L, _L)]
        rowsel[i // _PACK, pl.ds((i % _PACK) * _L, _L)] = v >> 3

    copies = []
    for j in range(_NCHUNK):
        copies.append(
            pltpu.async_copy(
                table_hbm.at[rowsel.at[j]],
                big_v.at[pl.ds(j * _CHUNK, _CHUNK)],
                sem,
            )
        )
    for c in copies:
        c.wait()

    lane = lax.iota(jnp.int32, _L)

    def softmax_block(blk, carry):
        iv = idx_flat[pl.ds(blk * _L, _L)]
        colb = (iv & (_PACK - 1)) * ACTIONS
        row_ids = blk * _L + lane
        cols = [
            plsc.load_gather(big_v, [row_ids, colb + j]) for j in range(ACTIONS)
        ]
        m = cols[0]
        for j in range(1, ACTIONS):
            m = jnp.maximum(m, cols[j])
        es = [jnp.exp(c - m) for c in cols]
        s = es[0]
        for j in range(1, ACTIONS):
            s = s + es[j]
        r = 1.0 / s
        orow = row_ids >> 3
        ocolb = (row_ids & (_PACK - 1)) * ACTIONS
        for j in range(ACTIONS):
            plsc.store_scatter(out_s, [orow, ocolb + j], es[j] * r)
        return carry

    lax.fori_loop(0, _B_PER_W // _L, softmax_block, 0)

    pltpu.sync_copy(out_s, out_hbm.at[pl.ds(wid * _OUT_PER_W, _OUT_PER_W)])


@jax.jit
def _run(x, table):
    tab128 = table.reshape(STATES // _PACK, _PACK * ACTIONS)
    mesh = plsc.VectorSubcoreMesh(core_axis_name="c", subcore_axis_name="s")
    kern = functools.partial(
        pl.kernel,
        out_type=jax.ShapeDtypeStruct((BATCH // _PACK, _PACK * ACTIONS),
                                      jnp.float32),
        mesh=mesh,
        scratch_types=[
            pltpu.VMEM((_B_PER_W,), jnp.int32),
            pltpu.VMEM((_NCHUNK, _CHUNK), jnp.int32),
            pltpu.VMEM((_B_PER_W, _PACK * ACTIONS), jnp.float32),
            pltpu.VMEM((_OUT_PER_W, _PACK * ACTIONS), jnp.float32),
            pltpu.SemaphoreType.DMA,
        ],
        compiler_params=pltpu.CompilerParams(needs_layout_passes=False),
    )(_sc_body)
    out = kern(x.astype(jnp.int32), tab128)
    return out.reshape(BATCH, ACTIONS)


def kernel(x, table):
    return _run(x, table)


# speedup vs baseline: 5.2959x; 5.2959x over previous
import functools

import jax
import jax.numpy as jnp
from jax import lax
from jax.experimental import pallas as pl
from jax.experimental.pallas import tpu as pltpu
from jax.experimental.pallas import tpu_sc as plsc

BATCH = 16384
ACTIONS = 16

_info = plsc.get_sparse_core_info()
_NC, _NS, _L = _info.num_cores, _info.num_subcores, _info.num_lanes
_NW = _NC * _NS
_B_PER_W = BATCH // _NW
_NGROUP = _B_PER_W // _L


def _sc_body(x_hbm, tab_hbm, out_hbm, idx_flat, blocks, outbuf, sem):
    wid = lax.axis_index("s") * _NC + lax.axis_index("c")
    base = wid * _B_PER_W

    pltpu.sync_copy(x_hbm.at[pl.ds(base, _B_PER_W)], idx_flat)

    lane = lax.iota(jnp.int32, _L)

    def group(g, carry):
        iv = idx_flat[pl.ds(g * _L, _L)]
        blk = iv >> 7
        col = iv & 127
        copies = []
        for k in range(_L):
            copies.append(
                pltpu.async_copy(
                    tab_hbm.at[:, pl.ds(blk[k] * 128, 128)],
                    blocks.at[k],
                    sem,
                )
            )
        for c in copies:
            c.wait()
        vs = [
            plsc.load_gather(blocks, [lane, jnp.full((_L,), j, jnp.int32), col])
            for j in range(ACTIONS)
        ]
        m = vs[0]
        for j in range(1, ACTIONS):
            m = jnp.maximum(m, vs[j])
        es = [jnp.exp(v - m) for v in vs]
        s = es[0]
        for j in range(1, ACTIONS):
            s = s + es[j]
        r = 1.0 / s
        opos = g * _L + lane
        for j in range(ACTIONS):
            plsc.store_scatter(
                outbuf, [jnp.full((_L,), j, jnp.int32), opos], es[j] * r
            )
        return carry

    lax.fori_loop(0, _NGROUP, group, 0)

    pltpu.sync_copy(outbuf, out_hbm.at[:, pl.ds(base, _B_PER_W)])


@jax.jit
def _run(x, table):
    tab_t = table.T
    mesh = plsc.VectorSubcoreMesh(core_axis_name="c", subcore_axis_name="s")
    kern = functools.partial(
        pl.kernel,
        out_type=jax.ShapeDtypeStruct((ACTIONS, BATCH), jnp.float32),
        mesh=mesh,
        scratch_types=[
            pltpu.VMEM((_B_PER_W,), jnp.int32),
            pltpu.VMEM((_L, ACTIONS, 128), jnp.float32),
            pltpu.VMEM((ACTIONS, _B_PER_W), jnp.float32),
            pltpu.SemaphoreType.DMA,
        ],
        compiler_params=pltpu.CompilerParams(needs_layout_passes=False),
    )(_sc_body)
    out = kern(x.astype(jnp.int32), tab_t)
    return out.T


def kernel(x, table):
    return _run(x, table)


# speedup vs baseline: 6.1852x; 1.1679x over previous
import functools

import jax
import jax.numpy as jnp
from jax import lax
from jax.experimental import pallas as pl
from jax.experimental.pallas import tpu as pltpu
from jax.experimental.pallas import tpu_sc as plsc

BATCH = 16384
ACTIONS = 16

_info = plsc.get_sparse_core_info()
_NC, _NS, _L = _info.num_cores, _info.num_subcores, _info.num_lanes
_NW = _NC * _NS
_B_PER_W = BATCH // _NW
_NGROUP = _B_PER_W // _L


def _sc_body(x_hbm, tab_hbm, out_hbm, idx_flat, blk0, blk1, outbuf,
             sem0, sem1):
    wid = lax.axis_index("s") * _NC + lax.axis_index("c")
    base = wid * _B_PER_W

    pltpu.sync_copy(x_hbm.at[pl.ds(base, _B_PER_W)], idx_flat)

    lane = lax.iota(jnp.int32, _L)

    def fire(g, blocks, sem):
        iv = idx_flat[pl.ds(g * _L, _L)]
        blk = iv >> 7
        for k in range(_L):
            pltpu.async_copy(
                tab_hbm.at[:, pl.ds(blk[k] * 128, 128)],
                blocks.at[k],
                sem,
            )

    def compute(g, blocks, sem):
        for k in range(_L):
            pltpu.make_async_copy(
                tab_hbm.at[:, pl.ds(0, 128)], blocks.at[k], sem
            ).wait()
        iv = idx_flat[pl.ds(g * _L, _L)]
        col = iv & 127
        vs = [
            plsc.load_gather(
                blocks, [lane, jnp.full((_L,), j, jnp.int32), col]
            )
            for j in range(ACTIONS)
        ]
        m = vs[0]
        for j in range(1, ACTIONS):
            m = jnp.maximum(m, vs[j])
        es = [jnp.exp(v - m) for v in vs]
        s = es[0]
        for j in range(1, ACTIONS):
            s = s + es[j]
        r = 1.0 / s
        opos = g * _L + lane
        for j in range(ACTIONS):
            plsc.store_scatter(
                outbuf, [jnp.full((_L,), j, jnp.int32), opos], es[j] * r
            )

    fire(0, blk0, sem0)

    def pair(p, carry):
        g = p * 2
        fire(g + 1, blk1, sem1)
        compute(g, blk0, sem0)

        @pl.when(g + 2 < _NGROUP)
        def _():
            fire(g + 2, blk0, sem0)

        compute(g + 1, blk1, sem1)
        return carry

    lax.fori_loop(0, _NGROUP // 2, pair, 0)

    pltpu.sync_copy(outbuf, out_hbm.at[:, pl.ds(base, _B_PER_W)])


@jax.jit
def _run(x, table):
    tab_t = table.T
    mesh = plsc.VectorSubcoreMesh(core_axis_name="c", subcore_axis_name="s")
    kern = functools.partial(
        pl.kernel,
        out_type=jax.ShapeDtypeStruct((ACTIONS, BATCH), jnp.float32),
        mesh=mesh,
        scratch_types=[
            pltpu.VMEM((_B_PER_W,), jnp.int32),
            pltpu.VMEM((_L, ACTIONS, 128), jnp.float32),
            pltpu.VMEM((_L, ACTIONS, 128), jnp.float32),
            pltpu.VMEM((ACTIONS, _B_PER_W), jnp.float32),
            pltpu.SemaphoreType.DMA,
            pltpu.SemaphoreType.DMA,
        ],
        compiler_params=pltpu.CompilerParams(needs_layout_passes=False),
    )(_sc_body)
    out = kern(x.astype(jnp.int32), tab_t)
    return out.T


def kernel(x, table):
    return _run(x, table)
